# baseline (speedup 1.0000x reference)
import functools

import jax
import jax.numpy as jnp
from jax import lax
from jax.experimental import pallas as pl
from jax.experimental.pallas import tpu as pltpu
from jax.experimental.pallas import tpu_sc as plsc

BATCH = 4096
SEQ = 200
D_MODEL = 64
D_PAD = 128
NUM_CORES = 2
NUM_SUBCORES = 16
NW = NUM_CORES * NUM_SUBCORES
BPW = BATCH // NW
IDX_PW = BPW * SEQ
NBUF = 2
LANES = 16

_mesh = plsc.VectorSubcoreMesh(core_axis_name="c", subcore_axis_name="s")


@functools.partial(
    pl.kernel,
    mesh=_mesh,
    compiler_params=pltpu.CompilerParams(needs_layout_passes=False),
    out_type=jax.ShapeDtypeStruct((SEQ, D_MODEL, BATCH), jnp.float32),
    scratch_types=[
        pltpu.VMEM((IDX_PW,), jnp.int32),
        pltpu.VMEM((SEQ, BPW), jnp.int32),
        pltpu.VMEM((NBUF, BPW, D_PAD), jnp.float32),
        pltpu.VMEM((NBUF, D_MODEL, BPW), jnp.float32),
        [pltpu.SemaphoreType.DMA] * NBUF,
        [pltpu.SemaphoreType.DMA] * NBUF,
    ],
)
def _embed_gather(x_hbm, table_hbm, out_hbm, idx_v, idx_t, rows_v, tr_v,
                  gsems, osems):
    wid = lax.axis_index("s") * NUM_CORES + lax.axis_index("c")
    lane0 = pl.multiple_of(wid * BPW, BPW)

    pltpu.sync_copy(x_hbm.at[pl.ds(wid * IDX_PW, IDX_PW)], idx_v)

    def tbody(s, carry):
        iota = lax.iota(jnp.int32, LANES)
        for jb in range(BPW // LANES):
            offs = (jb * LANES + iota) * SEQ + s
            v = plsc.load_gather(idx_v, [offs])
            idx_t[s, pl.ds(jb * LANES, LANES)] = v
        return carry

    lax.fori_loop(0, SEQ, tbody, 0)

    for b in range(NBUF):
        pltpu.async_copy(table_hbm.at[idx_t.at[b]], rows_v.at[b], gsems[b])

    def body(i, carry):
        g = i * NBUF
        for b in range(NBUF):
            s = g + b
            pltpu.make_async_copy(
                table_hbm.at[idx_t.at[s]], rows_v.at[b], gsems[b]
            ).wait()

            @pl.when(s >= NBUF)
            def _drain():
                pltpu.make_async_copy(
                    tr_v.at[b],
                    out_hbm.at[s - NBUF, :, pl.ds(lane0, BPW)],
                    osems[b],
                ).wait()

            def fbody(f, fcarry):
                iota = lax.iota(jnp.int32, LANES)
                fcol = jnp.broadcast_to(f, (LANES,))
                for jb in range(BPW // LANES):
                    v = plsc.load_gather(
                        rows_v.at[b], [jb * LANES + iota, fcol]
                    )
                    tr_v[b, f, pl.ds(jb * LANES, LANES)] = v
                return fcarry

            lax.fori_loop(0, D_MODEL, fbody, 0)

            pltpu.async_copy(
                tr_v.at[b], out_hbm.at[s, :, pl.ds(lane0, BPW)], osems[b]
            )

            @pl.when(s + NBUF < SEQ)
            def _issue():
                pltpu.async_copy(
                    table_hbm.at[idx_t.at[s + NBUF]], rows_v.at[b], gsems[b]
                )

        return carry

    lax.fori_loop(0, SEQ // NBUF, body, 0)

    for b in range(NBUF):
        pltpu.make_async_copy(
            tr_v.at[b],
            out_hbm.at[SEQ - NBUF + b, :, pl.ds(lane0, BPW)],
            osems[b],
        ).wait()


def kernel(x, table):
    table_pad = jnp.pad(table, ((0, 0), (0, D_PAD - D_MODEL)))
    out_t = _embed_gather(x.reshape(-1), table_pad)
    return jnp.transpose(out_t, (2, 0, 1))


# speedup vs baseline: 1.4879x; 1.4879x over previous
import functools

import jax
import jax.numpy as jnp
from jax import lax
from jax.experimental import pallas as pl
from jax.experimental.pallas import tpu as pltpu
from jax.experimental.pallas import tpu_sc as plsc

BATCH = 4096
SEQ = 200
D_MODEL = 64
D_PAD = 128
NUM_CORES = 2
NUM_SUBCORES = 16
NW = NUM_CORES * NUM_SUBCORES
BPW = BATCH // NW
IDX_PW = BPW * SEQ
NBUF = 2
LANES = 16

_mesh = plsc.VectorSubcoreMesh(core_axis_name="c", subcore_axis_name="s")


@functools.partial(
    pl.kernel,
    mesh=_mesh,
    compiler_params=pltpu.CompilerParams(needs_layout_passes=False),
    out_type=jax.ShapeDtypeStruct((SEQ, D_MODEL, BATCH), jnp.float32),
    scratch_types=[
        pltpu.VMEM((IDX_PW,), jnp.int32),
        pltpu.VMEM((SEQ, BPW), jnp.int32),
        pltpu.VMEM((NBUF, BPW, D_PAD), jnp.float32),
        pltpu.VMEM((NBUF, D_MODEL, BPW), jnp.float32),
        [pltpu.SemaphoreType.DMA] * NBUF,
        [pltpu.SemaphoreType.DMA] * NBUF,
    ],
)
def _embed_gather(x_hbm, table_hbm, out_hbm, idx_v, idx_t, rows_v, tr_v,
                  gsems, osems):
    wid = lax.axis_index("s") * NUM_CORES + lax.axis_index("c")
    lane0 = pl.multiple_of(wid * BPW, BPW)

    pltpu.sync_copy(x_hbm.at[pl.ds(wid * IDX_PW, IDX_PW)], idx_v)

    @plsc.parallel_loop(0, SEQ, unroll=4)
    def _idx_transpose(s):
        iota = lax.iota(jnp.int32, LANES)
        for jb in range(BPW // LANES):
            offs = (jb * LANES + iota) * SEQ + s
            v = plsc.load_gather(idx_v, [offs])
            idx_t[s, pl.ds(jb * LANES, LANES)] = v

    for b in range(NBUF):
        pltpu.async_copy(table_hbm.at[idx_t.at[b]], rows_v.at[b], gsems[b])

    def body(i, carry):
        g = i * NBUF
        for b in range(NBUF):
            s = g + b
            pltpu.make_async_copy(
                table_hbm.at[idx_t.at[s]], rows_v.at[b], gsems[b]
            ).wait()

            @pl.when(s >= NBUF)
            def _drain():
                pltpu.make_async_copy(
                    tr_v.at[b],
                    out_hbm.at[s - NBUF, :, pl.ds(lane0, BPW)],
                    osems[b],
                ).wait()

            @plsc.parallel_loop(0, D_MODEL, unroll=8)
            def _transpose(f):
                iota = lax.iota(jnp.int32, LANES)
                fcol = jnp.broadcast_to(f, (LANES,))
                for jb in range(BPW // LANES):
                    v = plsc.load_gather(
                        rows_v.at[b], [jb * LANES + iota, fcol]
                    )
                    tr_v[b, f, pl.ds(jb * LANES, LANES)] = v

            pltpu.async_copy(
                tr_v.at[b], out_hbm.at[s, :, pl.ds(lane0, BPW)], osems[b]
            )

            @pl.when(s + NBUF < SEQ)
            def _issue():
                pltpu.async_copy(
                    table_hbm.at[idx_t.at[s + NBUF]], rows_v.at[b], gsems[b]
                )

        return carry

    lax.fori_loop(0, SEQ // NBUF, body, 0)

    for b in range(NBUF):
        pltpu.make_async_copy(
            tr_v.at[b],
            out_hbm.at[SEQ - NBUF + b, :, pl.ds(lane0, BPW)],
            osems[b],
        ).wait()


def kernel(x, table):
    table_pad = jnp.pad(table, ((0, 0), (0, D_PAD - D_MODEL)))
    out_t = _embed_gather(x.reshape(-1), table_pad)
    return jnp.transpose(out_t, (2, 0, 1))


# speedup vs baseline: 1.7208x; 1.1566x over previous
import functools

import jax
import jax.numpy as jnp
from jax import lax
from jax.experimental import pallas as pl
from jax.experimental.pallas import tpu as pltpu
from jax.experimental.pallas import tpu_sc as plsc

BATCH = 4096
SEQ = 200
D_MODEL = 64
D_PAD = 128
VOCAB = 1000000
NUM_CORES = 2
NUM_SUBCORES = 16
NW = NUM_CORES * NUM_SUBCORES
BPW = BATCH // NW
IDX_PW = BPW * SEQ
NBUF = 2
LANES = 16

VTILES = VOCAB // D_PAD
PAIR_ROWS = VOCAB // 2
REM_PAIRS = (VOCAB - VTILES * D_PAD) // 2
A_SLOTS = ((VTILES + NW - 1) // NW + NBUF - 1) // NBUF * NBUF

_mesh = plsc.VectorSubcoreMesh(core_axis_name="c", subcore_axis_name="s")


@functools.partial(
    pl.kernel,
    mesh=_mesh,
    compiler_params=pltpu.CompilerParams(needs_layout_passes=False),
    out_type=jax.ShapeDtypeStruct((PAIR_ROWS, D_PAD), jnp.float32),
    scratch_types=[
        pltpu.VMEM((NBUF, D_MODEL, D_PAD), jnp.float32),
        pltpu.VMEM((NBUF, D_PAD // 2, D_PAD), jnp.float32),
        pltpu.VMEM((REM_PAIRS, D_PAD), jnp.float32),
        [pltpu.SemaphoreType.DMA] * NBUF,
        [pltpu.SemaphoreType.DMA] * NBUF,
    ],
)
def _table_pack(tt_hbm, rem_hbm, out_hbm, in_v, tr_v, rem_v, isems, osems):
    wid = lax.axis_index("s") * NUM_CORES + lax.axis_index("c")

    def lane0(it):
        return pl.multiple_of((wid + it * NW) * D_PAD, D_PAD)

    def prow0(it):
        return pl.multiple_of((wid + it * NW) * (D_PAD // 2), D_PAD // 2)

    for b in range(NBUF):
        pltpu.async_copy(
            tt_hbm.at[:, pl.ds(lane0(b), D_PAD)], in_v.at[b], isems[b]
        )

    def body(i, carry):
        g = i * NBUF
        for b in range(NBUF):
            it = g + b

            @pl.when(wid + it * NW < VTILES)
            def _work():
                pltpu.make_async_copy(
                    tt_hbm.at[:, pl.ds(lane0(it), D_PAD)], in_v.at[b],
                    isems[b],
                ).wait()

                @pl.when(it >= NBUF)
                def _drain():
                    pltpu.make_async_copy(
                        tr_v.at[b],
                        out_hbm.at[pl.ds(prow0(it - NBUF), D_PAD // 2)],
                        osems[b],
                    ).wait()

                @plsc.parallel_loop(0, D_PAD // LANES, unroll=1)
                def _pack(cb):
                    iota = lax.iota(jnp.int32, LANES)
                    c0 = cb * LANES
                    h = c0 // D_MODEL
                    f0 = lax.rem(c0, D_MODEL)
                    for pb in range(D_PAD // 2 // LANES):
                        p0 = pb * LANES
                        for k in range(LANES):
                            rot = (iota + k) & (LANES - 1)
                            v = plsc.load_gather(
                                in_v.at[b], [f0 + rot, 2 * (p0 + iota) + h]
                            )
                            plsc.store_scatter(
                                tr_v.at[b], [p0 + iota, c0 + rot], v
                            )

                pltpu.async_copy(
                    tr_v.at[b],
                    out_hbm.at[pl.ds(prow0(it), D_PAD // 2)],
                    osems[b],
                )

                @pl.when(wid + (it + NBUF) * NW < VTILES)
                def _issue():
                    pltpu.async_copy(
                        tt_hbm.at[:, pl.ds(lane0(it + NBUF), D_PAD)],
                        in_v.at[b],
                        isems[b],
                    )

        return carry

    lax.fori_loop(0, A_SLOTS // NBUF, body, 0)

    for b in range(NBUF):
        pltpu.make_async_copy(
            tr_v.at[b], out_hbm.at[pl.ds(prow0(b), D_PAD // 2)], osems[b]
        ).wait()

    @pl.when(wid == 0)
    def _rem():
        pltpu.sync_copy(rem_hbm, rem_v)
        pltpu.sync_copy(
            rem_v, out_hbm.at[pl.ds(VTILES * (D_PAD // 2), REM_PAIRS)]
        )


@functools.partial(
    pl.kernel,
    mesh=_mesh,
    compiler_params=pltpu.CompilerParams(needs_layout_passes=False),
    out_type=jax.ShapeDtypeStruct((SEQ, D_MODEL, BATCH), jnp.float32),
    scratch_types=[
        pltpu.VMEM((IDX_PW,), jnp.int32),
        pltpu.VMEM((SEQ, BPW), jnp.int32),
        pltpu.VMEM((SEQ, BPW), jnp.int32),
        pltpu.VMEM((NBUF, BPW, D_PAD), jnp.float32),
        pltpu.VMEM((NBUF, D_MODEL, BPW), jnp.float32),
        [pltpu.SemaphoreType.DMA] * NBUF,
        [pltpu.SemaphoreType.DMA] * NBUF,
    ],
)
def _embed_gather(x_hbm, table_hbm, out_hbm, idx_v, idx_t, par_t, rows_v,
                  tr_v, gsems, osems):
    wid = lax.axis_index("s") * NUM_CORES + lax.axis_index("c")
    lane_base = pl.multiple_of(wid * BPW, BPW)

    pltpu.sync_copy(x_hbm.at[pl.ds(wid * IDX_PW, IDX_PW)], idx_v)

    @plsc.parallel_loop(0, SEQ, unroll=4)
    def _idx_transpose(s):
        iota = lax.iota(jnp.int32, LANES)
        for jb in range(BPW // LANES):
            offs = (jb * LANES + iota) * SEQ + s
            v = plsc.load_gather(idx_v, [offs])
            idx_t[s, pl.ds(jb * LANES, LANES)] = v >> 1
            par_t[s, pl.ds(jb * LANES, LANES)] = (v & 1) << 6

    for b in range(NBUF):
        pltpu.async_copy(table_hbm.at[idx_t.at[b]], rows_v.at[b], gsems[b])

    def body(i, carry):
        g = i * NBUF
        for b in range(NBUF):
            s = g + b
            pltpu.make_async_copy(
                table_hbm.at[idx_t.at[s]], rows_v.at[b], gsems[b]
            ).wait()

            @pl.when(s >= NBUF)
            def _drain():
                pltpu.make_async_copy(
                    tr_v.at[b],
                    out_hbm.at[s - NBUF, :, pl.ds(lane_base, BPW)],
                    osems[b],
                ).wait()

            @plsc.parallel_loop(0, BPW // LANES, unroll=1)
            def _transpose(jb):
                iota = lax.iota(jnp.int32, LANES)
                j0 = pl.multiple_of(jb * LANES, LANES)
                rowsel = j0 + iota
                par16 = par_t[s, pl.ds(j0, LANES)]
                for fb in range(D_MODEL // LANES):
                    f0 = fb * LANES
                    for k in range(LANES):
                        rot = (iota + k) & (LANES - 1)
                        v = plsc.load_gather(
                            rows_v.at[b], [rowsel, par16 + (f0 + rot)]
                        )
                        plsc.store_scatter(
                            tr_v.at[b], [f0 + rot, rowsel], v
                        )

            pltpu.async_copy(
                tr_v.at[b], out_hbm.at[s, :, pl.ds(lane_base, BPW)], osems[b]
            )

            @pl.when(s + NBUF < SEQ)
            def _issue():
                pltpu.async_copy(
                    table_hbm.at[idx_t.at[s + NBUF]], rows_v.at[b], gsems[b]
                )

        return carry

    lax.fori_loop(0, SEQ // NBUF, body, 0)

    for b in range(NBUF):
        pltpu.make_async_copy(
            tr_v.at[b],
            out_hbm.at[SEQ - NBUF + b, :, pl.ds(lane_base, BPW)],
            osems[b],
        ).wait()


def kernel(x, table):
    tt = table.T
    rem = table[VTILES * D_PAD:].reshape(REM_PAIRS, D_PAD)
    table_pairs = _table_pack(tt, rem)
    out_t = _embed_gather(x.reshape(-1), table_pairs)
    return jnp.transpose(out_t, (2, 0, 1))
